# baseline (speedup 1.0000x reference)
import functools

import jax
import jax.numpy as jnp
from jax.experimental import pallas as pl
from jax.experimental.pallas import tpu as pltpu

_TOPK = 64
_SEARCH_ITERS = 26


def _enc_body(nw, wc_cols, topk,
              x_ref, w_ref, be_ref, z_ref, wb_ref, acc_ref, thr_ref):
    rb = pl.program_id(0)
    wc = pl.program_id(1)

    @pl.when(wc < nw)
    def _matmul():
        wb = w_ref[...].astype(jnp.bfloat16)

        @pl.when(rb == 0)
        def _copy_out():
            wb_ref[...] = wb

        prod = jax.lax.dot_general(
            x_ref[...], wb, (((1,), (1,)), ((), ())),
            preferred_element_type=jnp.float32)
        acc_ref[wc] = prod + be_ref[0, pl.ds(wc * wc_cols, wc_cols)][None, :]

    @pl.when(wc == nw - 1)
    def _search():
        rb_rows = acc_ref.shape[1]
        zero = jnp.zeros((rb_rows, 1), jnp.float32)

        def rowmax(j, m):
            return jnp.maximum(m, jnp.max(acc_ref[j], axis=1, keepdims=True))

        hi = jax.lax.fori_loop(0, nw, rowmax, zero)
        lo = zero

        def it(_, lh):
            lo, hi = lh
            mid = 0.5 * (lo + hi)

            def cchunk(j, c):
                return c + jnp.sum(
                    (acc_ref[j] >= mid).astype(jnp.float32),
                    axis=1, keepdims=True)

            cnt = jax.lax.fori_loop(0, nw, cchunk, zero)
            ge = cnt >= float(topk)
            return jnp.where(ge, mid, lo), jnp.where(ge, hi, mid)

        lo, hi = jax.lax.fori_loop(0, _SEARCH_ITERS, it, (lo, hi))
        thr_ref[...] = lo

    @pl.when(wc >= nw)
    def _write_z():
        a = acc_ref[wc - nw]
        z_ref[...] = jnp.where(a >= thr_ref[...],
                               jnp.maximum(a, 0.0), 0.0)


def _dec_body(nk, z_ref, w_ref, bd_ref, xh_ref, acc_ref):
    kc = pl.program_id(0)

    @pl.when(kc == 0)
    def _init():
        acc_ref[...] = jnp.zeros_like(acc_ref)

    zb = z_ref[...].astype(jnp.bfloat16)
    acc_ref[...] += jax.lax.dot_general(
        zb, w_ref[...], (((1,), (0,)), ((), ())),
        preferred_element_type=jnp.float32)

    @pl.when(kc == nk - 1)
    def _fin():
        xh_ref[...] = acc_ref[...] + bd_ref[...]


def _loss_body(denom, x_ref, xh_ref, out_ref):
    d = xh_ref[...] - x_ref[...]
    out_ref[...] = (jnp.sum(d * d) * (1.0 / denom)).reshape(1, 1)


def kernel(x, W_dec, b_enc, b_dec):
    B, T, D_IN = x.shape
    D_SAE = W_dec.shape[0]
    d_flat = T * D_IN

    xf = x.reshape(B, d_flat)
    xb = xf.astype(jnp.bfloat16)
    Wf = W_dec.reshape(D_SAE, d_flat)
    be2 = b_enc.reshape(1, D_SAE)
    bd2 = b_dec.reshape(1, d_flat)

    RB = min(512, B)
    WC = min(256, D_SAE)
    NW = D_SAE // WC
    NZ = NW

    z, Wb = pl.pallas_call(
        functools.partial(_enc_body, NW, WC, _TOPK),
        grid=(B // RB, NW + NZ),
        in_specs=[
            pl.BlockSpec((RB, d_flat), lambda rb, wc: (rb, 0)),
            pl.BlockSpec((WC, d_flat),
                         lambda rb, wc: (jnp.minimum(wc, NW - 1), 0)),
            pl.BlockSpec((1, D_SAE), lambda rb, wc: (0, 0)),
        ],
        out_specs=[
            pl.BlockSpec(
                (RB, WC), lambda rb, wc: (rb, jnp.maximum(wc - NW, 0))),
            pl.BlockSpec(
                (WC, d_flat),
                lambda rb, wc: (jnp.where(rb == 0,
                                          jnp.minimum(wc, NW - 1),
                                          NW - 1), 0)),
        ],
        out_shape=[
            jax.ShapeDtypeStruct((B, D_SAE), jnp.float32),
            jax.ShapeDtypeStruct((D_SAE, d_flat), jnp.bfloat16),
        ],
        scratch_shapes=[
            pltpu.VMEM((NW, RB, WC), jnp.float32),
            pltpu.VMEM((RB, 1), jnp.float32),
        ],
    )(xb, Wf, be2)

    KC = min(1024, D_SAE)
    NK = D_SAE // KC
    xh = pl.pallas_call(
        functools.partial(_dec_body, NK),
        grid=(NK,),
        in_specs=[
            pl.BlockSpec((B, KC), lambda kc: (0, kc)),
            pl.BlockSpec((KC, d_flat), lambda kc: (kc, 0)),
            pl.BlockSpec((1, d_flat), lambda kc: (0, 0)),
        ],
        out_specs=pl.BlockSpec((B, d_flat), lambda kc: (0, 0)),
        out_shape=jax.ShapeDtypeStruct((B, d_flat), jnp.float32),
        scratch_shapes=[pltpu.VMEM((B, d_flat), jnp.float32)],
    )(z, Wb, bd2)

    lossm = pl.pallas_call(
        functools.partial(_loss_body, float(B * T)),
        grid=(1,),
        in_specs=[
            pl.BlockSpec((B, d_flat), lambda i: (0, 0)),
            pl.BlockSpec((B, d_flat), lambda i: (0, 0)),
        ],
        out_specs=pl.BlockSpec((1, 1), lambda i: (0, 0)),
        out_shape=jax.ShapeDtypeStruct((1, 1), jnp.float32),
    )(xf, xh)

    return (lossm[0, 0], xh.reshape(B, T, D_IN), z)


# speedup vs baseline: 1.3414x; 1.3414x over previous
import functools

import jax
import jax.numpy as jnp
from jax.experimental import pallas as pl
from jax.experimental.pallas import tpu as pltpu

_TOPK = 64
_SEARCH_ITERS = 26


def _enc_body(nw, wc_cols, topk,
              x_ref, w_ref, be_ref, z_ref, acc_ref, thr_ref):
    wc = pl.program_id(1)

    @pl.when(wc < nw)
    def _matmul():
        prod = jax.lax.dot_general(
            x_ref[...], w_ref[...], (((1,), (1,)), ((), ())),
            preferred_element_type=jnp.float32)
        acc_ref[wc] = prod + be_ref[0, pl.ds(wc * wc_cols, wc_cols)][None, :]

    @pl.when(wc == nw - 1)
    def _search():
        rb_rows = acc_ref.shape[1]
        zero = jnp.zeros((rb_rows, 1), jnp.float32)

        def rowmax(j, m):
            return jnp.maximum(m, jnp.max(acc_ref[j], axis=1, keepdims=True))

        hi = jax.lax.fori_loop(0, nw, rowmax, zero)
        lo = zero

        def it(_, lh):
            lo, hi = lh
            mid = 0.5 * (lo + hi)

            def cchunk(j, c):
                return c + jnp.sum(
                    (acc_ref[j] >= mid).astype(jnp.float32),
                    axis=1, keepdims=True)

            cnt = jax.lax.fori_loop(0, nw, cchunk, zero)
            ge = cnt >= float(topk)
            return jnp.where(ge, mid, lo), jnp.where(ge, hi, mid)

        lo, hi = jax.lax.fori_loop(0, _SEARCH_ITERS, it, (lo, hi))
        thr_ref[...] = lo

    @pl.when(wc >= nw)
    def _write_z():
        a = acc_ref[wc - nw]
        z_ref[...] = jnp.where(a >= thr_ref[...],
                               jnp.maximum(a, 0.0), 0.0)


def _dec_body(nk, z_ref, w_ref, bd_ref, xh_ref, acc_ref):
    kc = pl.program_id(0)

    @pl.when(kc == 0)
    def _init():
        acc_ref[...] = jnp.zeros_like(acc_ref)

    zb = z_ref[...].astype(jnp.bfloat16)
    acc_ref[...] += jax.lax.dot_general(
        zb, w_ref[...], (((1,), (0,)), ((), ())),
        preferred_element_type=jnp.float32)

    @pl.when(kc == nk - 1)
    def _fin():
        xh_ref[...] = acc_ref[...] + bd_ref[...]


def _loss_body(denom, x_ref, xh_ref, out_ref):
    d = xh_ref[...] - x_ref[...]
    out_ref[...] = (jnp.sum(d * d) * (1.0 / denom)).reshape(1, 1)


def kernel(x, W_dec, b_enc, b_dec):
    B, T, D_IN = x.shape
    D_SAE = W_dec.shape[0]
    d_flat = T * D_IN

    xf = x.reshape(B, d_flat)
    xb = xf.astype(jnp.bfloat16)
    Wb = W_dec.reshape(D_SAE, d_flat).astype(jnp.bfloat16)
    be2 = b_enc.reshape(1, D_SAE)
    bd2 = b_dec.reshape(1, d_flat)

    RB = min(512, B)
    WC = min(512, D_SAE)
    NW = D_SAE // WC
    NZ = NW

    z = pl.pallas_call(
        functools.partial(_enc_body, NW, WC, _TOPK),
        grid=(B // RB, NW + NZ),
        in_specs=[
            pl.BlockSpec((RB, d_flat), lambda rb, wc: (rb, 0)),
            pl.BlockSpec((WC, d_flat),
                         lambda rb, wc: (jnp.minimum(wc, NW - 1), 0)),
            pl.BlockSpec((1, D_SAE), lambda rb, wc: (0, 0)),
        ],
        out_specs=pl.BlockSpec(
            (RB, WC), lambda rb, wc: (rb, jnp.maximum(wc - NW, 0))),
        out_shape=jax.ShapeDtypeStruct((B, D_SAE), jnp.float32),
        scratch_shapes=[
            pltpu.VMEM((NW, RB, WC), jnp.float32),
            pltpu.VMEM((RB, 1), jnp.float32),
        ],
    )(xb, Wb, be2)

    KC = min(1024, D_SAE)
    NK = D_SAE // KC
    xh = pl.pallas_call(
        functools.partial(_dec_body, NK),
        grid=(NK,),
        in_specs=[
            pl.BlockSpec((B, KC), lambda kc: (0, kc)),
            pl.BlockSpec((KC, d_flat), lambda kc: (kc, 0)),
            pl.BlockSpec((1, d_flat), lambda kc: (0, 0)),
        ],
        out_specs=pl.BlockSpec((B, d_flat), lambda kc: (0, 0)),
        out_shape=jax.ShapeDtypeStruct((B, d_flat), jnp.float32),
        scratch_shapes=[pltpu.VMEM((B, d_flat), jnp.float32)],
    )(z, Wb, bd2)

    lossm = pl.pallas_call(
        functools.partial(_loss_body, float(B * T)),
        grid=(1,),
        in_specs=[
            pl.BlockSpec((B, d_flat), lambda i: (0, 0)),
            pl.BlockSpec((B, d_flat), lambda i: (0, 0)),
        ],
        out_specs=pl.BlockSpec((1, 1), lambda i: (0, 0)),
        out_shape=jax.ShapeDtypeStruct((1, 1), jnp.float32),
    )(xf, xh)

    return (lossm[0, 0], xh.reshape(B, T, D_IN), z)
